# baseline (speedup 1.0000x reference)
import jax
import jax.numpy as jnp
from jax import lax
from jax.experimental import pallas as pl
from jax.experimental.pallas import tpu as pltpu
from jax.experimental.pallas import tpu_sc as plsc

N_USERS = 25000
N_ITEMS = 25000
N_NODES = N_USERS + N_ITEMS
DIM = 64
N_LAYERS = 3

NC = 2
NS = 16
LANES = 16

N_HALF = 25088
NPT = N_HALF // NS
N_PAD = NC * N_HALF
N_ZPAD = N_PAD - N_NODES
CHUNK = 128
ACC_ROWS = N_HALF + CHUNK
ZPT = ACC_ROWS // NS
RB = 112

_MESH = plsc.VectorSubcoreMesh(
    core_axis_name="c", subcore_axis_name="s", num_cores=NC, num_subcores=NS
)
_PARAMS = pltpu.CompilerParams(use_tc_tiling_on_sc=False)
_F32 = jnp.float32
_I32 = jnp.int32


def _newton_rsqrt(v):
    i = lax.bitcast_convert_type(v, _I32)
    i = 0x5F3759DF - lax.shift_right_arithmetic(i, 1)
    y = lax.bitcast_convert_type(i, _F32)
    for _ in range(4):
        y = y * (1.5 - 0.5 * v * y * y)
    return y


def _route(colv, liv, base):
    for g in range(CHUNK // LANES):
        c16 = colv[pl.ds(g * LANES, LANES)]
        li = c16 - base
        ok = (li >= 0) & (li < N_HALF)
        tr = N_HALF + jnp.bitwise_and(c16, CHUNK - 1)
        liv[pl.ds(g * LANES, LANES)] = jnp.where(ok, li, tr)


def _deg_body(col_hbm, x0_hbm, dis_hbm, xs0_hbm,
              dacc, colv0, colv1, liv0, liv1, ones, z16, dav, xv, d16v,
              ssem0, ssem1):
    cid = lax.axis_index("c")
    sid = lax.axis_index("s")
    base = cid * N_HALF
    colv = (colv0, colv1)
    liv = (liv0, liv1)
    ssem = (ssem0, ssem1)

    one = jnp.full((LANES,), 1.0, _F32)
    zero = jnp.zeros((LANES,), _F32)

    def init_bufs(r, _):
        ones[r, :] = one
        z16[r, :] = zero
        return 0

    lax.fori_loop(0, CHUNK, init_bufs, 0)

    z0 = sid * ZPT
    for i in range(12):
        pltpu.sync_copy(z16, dacc.at[pl.ds(z0 + i * CHUNK, CHUNK)])
    pltpu.sync_copy(z16.at[pl.ds(0, ZPT - 12 * CHUNK)],
                    dacc.at[pl.ds(z0 + 12 * CHUNK, ZPT - 12 * CHUNK)])
    plsc.subcore_barrier()

    cpt = (col_hbm.shape[0] - 2 * CHUNK) // (NS * CHUNK)
    e0 = sid * cpt * CHUNK

    def stage(b, off):
        pltpu.sync_copy(col_hbm.at[pl.ds(off, CHUNK)], colv[b])
        _route(colv[b], liv[b], base)

    stage(0, e0)

    def pair(jj, _):
        j0 = jj * 2
        for b in range(2):
            j = j0 + b
            pltpu.async_copy(ones, dacc.at[liv[b]], ssem[b], add=True)
            stage(1 - b, e0 + (j + 1) * CHUNK)
            pltpu.make_async_copy(ones, dacc.at[liv[b]], ssem[b]).wait()
        return 0

    lax.fori_loop(0, cpt // 2, pair, 0)
    plsc.subcore_barrier()

    l0 = sid * NPT
    g0 = base + l0

    def ep_body(ci, _):
        r0 = ci * RB
        pltpu.sync_copy(dacc.at[pl.ds(l0 + r0, RB)], dav)
        pltpu.sync_copy(x0_hbm.at[pl.ds(g0 + r0, RB)], xv)

        def row(r, _):
            v = dav[r, :]
            y = _newton_rsqrt(v)
            y = jnp.where(v > 0.5, y, 0.0)
            d16v[r, :] = y
            for k in range(DIM // LANES):
                xv[r, pl.ds(k * LANES, LANES)] = (
                    xv[r, pl.ds(k * LANES, LANES)] * y
                )
            return 0

        lax.fori_loop(0, RB, row, 0)
        pltpu.sync_copy(d16v, dis_hbm.at[pl.ds(g0 + r0, RB)])
        pltpu.sync_copy(xv, xs0_hbm.at[pl.ds(g0 + r0, RB)])
        return 0

    lax.fori_loop(0, NPT // RB, ep_body, 0)


def _layer_body(row_hbm, col_hbm, xs_hbm, dis_hbm, y_hbm, xs2_hbm,
                acc, colv0, colv1, rowv0, rowv1, liv0, liv1, rows0, rows1,
                d16v, gsem0, gsem1, ssem0, ssem1):
    cid = lax.axis_index("c")
    sid = lax.axis_index("s")
    base = cid * N_HALF
    colv = (colv0, colv1)
    rowv = (rowv0, rowv1)
    liv = (liv0, liv1)
    rows = (rows0, rows1)
    gsem = (gsem0, gsem1)
    ssem = (ssem0, ssem1)

    z0 = sid * ZPT
    for i in range(8):
        pltpu.sync_copy(
            xs_hbm.at[pl.ds(N_NODES, N_ZPAD)],
            acc.at[pl.ds(z0 + i * N_ZPAD, N_ZPAD)],
        )
    pltpu.sync_copy(
        xs_hbm.at[pl.ds(N_NODES, ZPT - 8 * N_ZPAD)],
        acc.at[pl.ds(z0 + 8 * N_ZPAD, ZPT - 8 * N_ZPAD)],
    )
    plsc.subcore_barrier()

    cpt = (row_hbm.shape[0] - 2 * CHUNK) // (NS * CHUNK)
    e0 = sid * cpt * CHUNK

    def stage(b, off):
        pltpu.sync_copy(row_hbm.at[pl.ds(off, CHUNK)], rowv[b])
        pltpu.sync_copy(col_hbm.at[pl.ds(off, CHUNK)], colv[b])
        _route(colv[b], liv[b], base)
        pltpu.async_copy(xs_hbm.at[rowv[b]], rows[b], gsem[b])

    for b in range(2):
        stage(b, e0 + b * CHUNK)

    def pair(jj, _):
        j0 = jj * 2
        for b in range(2):
            j = j0 + b
            pltpu.make_async_copy(xs_hbm.at[rowv[b]], rows[b], gsem[b]).wait()
            pltpu.async_copy(rows[b], acc.at[liv[b]], ssem[b], add=True)
            pltpu.make_async_copy(rows[b], acc.at[liv[b]], ssem[b]).wait()
            stage(b, e0 + (j + 2) * CHUNK)
        return 0

    lax.fori_loop(0, cpt // 2, pair, 0)

    for b in range(2):
        pltpu.make_async_copy(xs_hbm.at[rowv[b]], rows[b], gsem[b]).wait()
    plsc.subcore_barrier()

    l0 = sid * NPT
    g0 = base + l0
    buf = rows0.at[pl.ds(0, RB)]

    def ep_body(ci, _):
        r0 = ci * RB
        pltpu.sync_copy(acc.at[pl.ds(l0 + r0, RB)], buf)
        pltpu.sync_copy(dis_hbm.at[pl.ds(g0 + r0, RB)], d16v)

        def row_y(r, _):
            v = d16v[r, :]
            for k in range(DIM // LANES):
                rows0[r, pl.ds(k * LANES, LANES)] = (
                    rows0[r, pl.ds(k * LANES, LANES)] * v
                )
            return 0

        lax.fori_loop(0, RB, row_y, 0)
        pltpu.sync_copy(buf, y_hbm.at[pl.ds(g0 + r0, RB)])
        lax.fori_loop(0, RB, row_y, 0)
        pltpu.sync_copy(buf, xs2_hbm.at[pl.ds(g0 + r0, RB)])
        return 0

    lax.fori_loop(0, NPT // RB, ep_body, 0)


_deg_call = pl.kernel(
    _deg_body,
    out_type=(
        jax.ShapeDtypeStruct((N_PAD, LANES), _F32),
        jax.ShapeDtypeStruct((N_PAD, DIM), _F32),
    ),
    mesh=_MESH,
    compiler_params=_PARAMS,
    scratch_types=[
        pltpu.VMEM_SHARED((ACC_ROWS, LANES), _F32),
        pltpu.VMEM((CHUNK,), _I32),
        pltpu.VMEM((CHUNK,), _I32),
        pltpu.VMEM((CHUNK,), _I32),
        pltpu.VMEM((CHUNK,), _I32),
        pltpu.VMEM((CHUNK, LANES), _F32),
        pltpu.VMEM((CHUNK, LANES), _F32),
        pltpu.VMEM((RB, LANES), _F32),
        pltpu.VMEM((RB, DIM), _F32),
        pltpu.VMEM((RB, LANES), _F32),
        pltpu.SemaphoreType.DMA,
        pltpu.SemaphoreType.DMA,
    ],
    name="lightgcn_deg_sc",
)

_layer_call = pl.kernel(
    _layer_body,
    out_type=(
        jax.ShapeDtypeStruct((N_PAD, DIM), _F32),
        jax.ShapeDtypeStruct((N_PAD, DIM), _F32),
    ),
    mesh=_MESH,
    compiler_params=_PARAMS,
    scratch_types=[
        pltpu.VMEM_SHARED((ACC_ROWS, DIM), _F32),
        pltpu.VMEM((CHUNK,), _I32),
        pltpu.VMEM((CHUNK,), _I32),
        pltpu.VMEM((CHUNK,), _I32),
        pltpu.VMEM((CHUNK,), _I32),
        pltpu.VMEM((CHUNK,), _I32),
        pltpu.VMEM((CHUNK,), _I32),
        pltpu.VMEM((CHUNK, DIM), _F32),
        pltpu.VMEM((CHUNK, DIM), _F32),
        pltpu.VMEM((RB, LANES), _F32),
        pltpu.SemaphoreType.DMA,
        pltpu.SemaphoreType.DMA,
        pltpu.SemaphoreType.DMA,
        pltpu.SemaphoreType.DMA,
    ],
    name="lightgcn_layer_sc",
)


def _avg_body(a, b, c, d, o):
    o[...] = 0.25 * (a[...] + b[...] + c[...] + d[...])


_AVG_BLK = 1024


@jax.jit
def _pipeline(x0p, r2, c2):
    dis, xs = _deg_call(c2, x0p)
    ys = []
    for _ in range(N_LAYERS):
        y, xs = _layer_call(r2, c2, xs, dis)
        ys.append(y)
    spec = pl.BlockSpec((_AVG_BLK, DIM), lambda i: (i, 0))
    final = pl.pallas_call(
        _avg_body,
        grid=(N_PAD // _AVG_BLK,),
        in_specs=[spec] * 4,
        out_specs=spec,
        out_shape=jax.ShapeDtypeStruct((N_PAD, DIM), _F32),
    )(x0p, ys[0], ys[1], ys[2])
    return final


def kernel(user_embedding_weight, item_embedding_weight, edge_index):
    x0 = jnp.concatenate([user_embedding_weight, item_embedding_weight], axis=0)
    x0p = jnp.pad(x0, ((0, N_PAD - N_NODES), (0, 0)))
    ei = edge_index.astype(_I32)
    e2 = 2 * ei.shape[1]
    unit = NS * CHUNK
    n_chunks = (e2 + unit - 1) // unit
    n_chunks += n_chunks % 2
    e2p = unit * n_chunks + 2 * CHUNK
    r2 = jnp.concatenate([ei[0], ei[1]])
    c2 = jnp.concatenate([ei[1], ei[0]])
    r2 = jnp.pad(r2, (0, e2p - e2))
    c2 = jnp.pad(c2, (0, e2p - e2), constant_values=-1)
    final = _pipeline(x0p, r2, c2)
    return final[:N_NODES]


# speedup vs baseline: 1.1595x; 1.1595x over previous
import jax
import jax.numpy as jnp
from jax import lax
from jax.experimental import pallas as pl
from jax.experimental.pallas import tpu as pltpu
from jax.experimental.pallas import tpu_sc as plsc

N_USERS = 25000
N_ITEMS = 25000
N_NODES = N_USERS + N_ITEMS
DIM = 64
N_LAYERS = 3

NC = 2
NS = 16
LANES = 16

N_HALF = 25088
NPT = N_HALF // NS
N_PAD = NC * N_HALF
N_ZPAD = N_PAD - N_NODES
CHUNK = 128
ACC_ROWS = N_HALF + CHUNK
ZPT = ACC_ROWS // NS
RB = 112

_MESH = plsc.VectorSubcoreMesh(
    core_axis_name="c", subcore_axis_name="s", num_cores=NC, num_subcores=NS
)
_PARAMS = pltpu.CompilerParams(use_tc_tiling_on_sc=False)
_F32 = jnp.float32
_I32 = jnp.int32


def _newton_rsqrt(v):
    i = lax.bitcast_convert_type(v, _I32)
    i = 0x5F3759DF - lax.shift_right_arithmetic(i, 1)
    y = lax.bitcast_convert_type(i, _F32)
    for _ in range(4):
        y = y * (1.5 - 0.5 * v * y * y)
    return y


def _route(colv, liv, base):
    for g in range(CHUNK // LANES):
        c16 = colv[pl.ds(g * LANES, LANES)]
        li = c16 - base
        ok = (li >= 0) & (li < N_HALF)
        tr = N_HALF + jnp.bitwise_and(c16, CHUNK - 1)
        liv[pl.ds(g * LANES, LANES)] = jnp.where(ok, li, tr)


def _deg_body(col_hbm, x0_hbm, dis_hbm, xs0_hbm,
              dacc, colv0, colv1, liv0, liv1, ones, z16, dav, xv, d16v,
              ssem0, ssem1):
    cid = lax.axis_index("c")
    sid = lax.axis_index("s")
    base = cid * N_HALF
    colv = (colv0, colv1)
    liv = (liv0, liv1)
    ssem = (ssem0, ssem1)

    one = jnp.full((LANES,), 1.0, _F32)
    zero = jnp.zeros((LANES,), _F32)

    def init_bufs(r, _):
        ones[r, :] = one
        z16[r, :] = zero
        return 0

    lax.fori_loop(0, CHUNK, init_bufs, 0)

    z0 = sid * ZPT
    for i in range(12):
        pltpu.sync_copy(z16, dacc.at[pl.ds(z0 + i * CHUNK, CHUNK)])
    pltpu.sync_copy(z16.at[pl.ds(0, ZPT - 12 * CHUNK)],
                    dacc.at[pl.ds(z0 + 12 * CHUNK, ZPT - 12 * CHUNK)])
    plsc.subcore_barrier()

    cpt = (col_hbm.shape[0] - 2 * CHUNK) // (NS * CHUNK)
    e0 = sid * cpt * CHUNK

    def stage(b, off):
        pltpu.sync_copy(col_hbm.at[pl.ds(off, CHUNK)], colv[b])
        _route(colv[b], liv[b], base)

    stage(0, e0)

    def pair(jj, _):
        j0 = jj * 2
        for b in range(2):
            j = j0 + b
            pltpu.async_copy(ones, dacc.at[liv[b]], ssem[b], add=True)
            stage(1 - b, e0 + (j + 1) * CHUNK)
            pltpu.make_async_copy(ones, dacc.at[liv[b]], ssem[b]).wait()
        return 0

    lax.fori_loop(0, cpt // 2, pair, 0)
    plsc.subcore_barrier()

    l0 = sid * NPT
    g0 = base + l0

    def ep_body(ci, _):
        r0 = ci * RB
        pltpu.sync_copy(dacc.at[pl.ds(l0 + r0, RB)], dav)
        pltpu.sync_copy(x0_hbm.at[pl.ds(g0 + r0, RB)], xv)

        def row(r, _):
            v = dav[r, :]
            y = _newton_rsqrt(v)
            y = jnp.where(v > 0.5, y, 0.0)
            d16v[r, :] = y
            for k in range(DIM // LANES):
                xv[r, pl.ds(k * LANES, LANES)] = (
                    xv[r, pl.ds(k * LANES, LANES)] * y
                )
            return 0

        lax.fori_loop(0, RB, row, 0)
        pltpu.sync_copy(d16v, dis_hbm.at[pl.ds(g0 + r0, RB)])
        pltpu.sync_copy(xv, xs0_hbm.at[pl.ds(g0 + r0, RB)])
        return 0

    lax.fori_loop(0, NPT // RB, ep_body, 0)


def _layer_body(row_hbm, col_hbm, xs_hbm, dis_hbm, y_hbm, xs2_hbm,
                acc, colv0, colv1, rowv0, rowv1, liv0, liv1, rows0, rows1,
                d16v, gsem0, gsem1, ssem0, ssem1):
    cid = lax.axis_index("c")
    sid = lax.axis_index("s")
    base = cid * N_HALF
    colv = (colv0, colv1)
    rowv = (rowv0, rowv1)
    liv = (liv0, liv1)
    rows = (rows0, rows1)
    gsem = (gsem0, gsem1)
    ssem = (ssem0, ssem1)

    z0 = sid * ZPT
    for i in range(8):
        pltpu.sync_copy(
            xs_hbm.at[pl.ds(N_NODES, N_ZPAD)],
            acc.at[pl.ds(z0 + i * N_ZPAD, N_ZPAD)],
        )
    pltpu.sync_copy(
        xs_hbm.at[pl.ds(N_NODES, ZPT - 8 * N_ZPAD)],
        acc.at[pl.ds(z0 + 8 * N_ZPAD, ZPT - 8 * N_ZPAD)],
    )
    plsc.subcore_barrier()

    cpt = (row_hbm.shape[0] - 2 * CHUNK) // (NS * CHUNK)
    e0 = sid * cpt * CHUNK

    for b in range(2):
        off = e0 + b * CHUNK
        pltpu.sync_copy(row_hbm.at[pl.ds(off, CHUNK)], rowv[b])
        pltpu.sync_copy(col_hbm.at[pl.ds(off, CHUNK)], colv[b])
        _route(colv[b], liv[b], base)
        pltpu.async_copy(xs_hbm.at[rowv[b]], rows[b], gsem[b])

    def pair(jj, _):
        j0 = jj * 2
        for b in range(2):
            j = j0 + b
            pltpu.make_async_copy(xs_hbm.at[rowv[b]], rows[b], gsem[b]).wait()
            pltpu.async_copy(rows[b], acc.at[liv[b]], ssem[b], add=True)
            off = e0 + (j + 2) * CHUNK
            pltpu.sync_copy(row_hbm.at[pl.ds(off, CHUNK)], rowv[b])
            pltpu.sync_copy(col_hbm.at[pl.ds(off, CHUNK)], colv[b])
            pltpu.make_async_copy(rows[b], acc.at[liv[b]], ssem[b]).wait()
            _route(colv[b], liv[b], base)
            pltpu.async_copy(xs_hbm.at[rowv[b]], rows[b], gsem[b])
        return 0

    lax.fori_loop(0, cpt // 2, pair, 0)

    for b in range(2):
        pltpu.make_async_copy(xs_hbm.at[rowv[b]], rows[b], gsem[b]).wait()
    plsc.subcore_barrier()

    l0 = sid * NPT
    g0 = base + l0
    buf = rows0.at[pl.ds(0, RB)]

    def ep_body(ci, _):
        r0 = ci * RB
        pltpu.sync_copy(acc.at[pl.ds(l0 + r0, RB)], buf)
        pltpu.sync_copy(dis_hbm.at[pl.ds(g0 + r0, RB)], d16v)

        def row_y(r, _):
            v = d16v[r, :]
            for k in range(DIM // LANES):
                rows0[r, pl.ds(k * LANES, LANES)] = (
                    rows0[r, pl.ds(k * LANES, LANES)] * v
                )
            return 0

        lax.fori_loop(0, RB, row_y, 0)
        pltpu.sync_copy(buf, y_hbm.at[pl.ds(g0 + r0, RB)])
        lax.fori_loop(0, RB, row_y, 0)
        pltpu.sync_copy(buf, xs2_hbm.at[pl.ds(g0 + r0, RB)])
        return 0

    lax.fori_loop(0, NPT // RB, ep_body, 0)


_deg_call = pl.kernel(
    _deg_body,
    out_type=(
        jax.ShapeDtypeStruct((N_PAD, LANES), _F32),
        jax.ShapeDtypeStruct((N_PAD, DIM), _F32),
    ),
    mesh=_MESH,
    compiler_params=_PARAMS,
    scratch_types=[
        pltpu.VMEM_SHARED((ACC_ROWS, LANES), _F32),
        pltpu.VMEM((CHUNK,), _I32),
        pltpu.VMEM((CHUNK,), _I32),
        pltpu.VMEM((CHUNK,), _I32),
        pltpu.VMEM((CHUNK,), _I32),
        pltpu.VMEM((CHUNK, LANES), _F32),
        pltpu.VMEM((CHUNK, LANES), _F32),
        pltpu.VMEM((RB, LANES), _F32),
        pltpu.VMEM((RB, DIM), _F32),
        pltpu.VMEM((RB, LANES), _F32),
        pltpu.SemaphoreType.DMA,
        pltpu.SemaphoreType.DMA,
    ],
    name="lightgcn_deg_sc",
)

_layer_call = pl.kernel(
    _layer_body,
    out_type=(
        jax.ShapeDtypeStruct((N_PAD, DIM), _F32),
        jax.ShapeDtypeStruct((N_PAD, DIM), _F32),
    ),
    mesh=_MESH,
    compiler_params=_PARAMS,
    scratch_types=[
        pltpu.VMEM_SHARED((ACC_ROWS, DIM), _F32),
        pltpu.VMEM((CHUNK,), _I32),
        pltpu.VMEM((CHUNK,), _I32),
        pltpu.VMEM((CHUNK,), _I32),
        pltpu.VMEM((CHUNK,), _I32),
        pltpu.VMEM((CHUNK,), _I32),
        pltpu.VMEM((CHUNK,), _I32),
        pltpu.VMEM((CHUNK, DIM), _F32),
        pltpu.VMEM((CHUNK, DIM), _F32),
        pltpu.VMEM((RB, LANES), _F32),
        pltpu.SemaphoreType.DMA,
        pltpu.SemaphoreType.DMA,
        pltpu.SemaphoreType.DMA,
        pltpu.SemaphoreType.DMA,
    ],
    name="lightgcn_layer_sc",
)


def _avg_body(a, b, c, d, o):
    o[...] = 0.25 * (a[...] + b[...] + c[...] + d[...])


_AVG_BLK = 1024


@jax.jit
def _pipeline(x0p, r2, c2):
    dis, xs = _deg_call(c2, x0p)
    ys = []
    for _ in range(N_LAYERS):
        y, xs = _layer_call(r2, c2, xs, dis)
        ys.append(y)
    spec = pl.BlockSpec((_AVG_BLK, DIM), lambda i: (i, 0))
    final = pl.pallas_call(
        _avg_body,
        grid=(N_PAD // _AVG_BLK,),
        in_specs=[spec] * 4,
        out_specs=spec,
        out_shape=jax.ShapeDtypeStruct((N_PAD, DIM), _F32),
    )(x0p, ys[0], ys[1], ys[2])
    return final


def kernel(user_embedding_weight, item_embedding_weight, edge_index):
    x0 = jnp.concatenate([user_embedding_weight, item_embedding_weight], axis=0)
    x0p = jnp.pad(x0, ((0, N_PAD - N_NODES), (0, 0)))
    ei = edge_index.astype(_I32)
    e2 = 2 * ei.shape[1]
    unit = NS * CHUNK
    n_chunks = (e2 + unit - 1) // unit
    n_chunks += n_chunks % 2
    e2p = unit * n_chunks + 2 * CHUNK
    r2 = jnp.concatenate([ei[0], ei[1]])
    c2 = jnp.concatenate([ei[1], ei[0]])
    r2 = jnp.pad(r2, (0, e2p - e2))
    c2 = jnp.pad(c2, (0, e2p - e2), constant_values=-1)
    final = _pipeline(x0p, r2, c2)
    return final[:N_NODES]
